# baseline (speedup 1.0000x reference)
import functools

import jax
import jax.numpy as jnp
from jax import lax
from jax.experimental import pallas as pl
from jax.experimental.pallas import tpu as pltpu
from jax.experimental.pallas import tpu_sc as plsc

D_V = 1_000_000
D_M = 64
B = 16384

NC = 2
NS = 16
NW = NC * NS
SEG = B // NW


HROWS = D_V // 4


def _body(idx_hbm, ta_hbm, tb_hbm, out_hbm, idx_v, a_ids_v, b_ids_v,
          rows_v, sem, gsem):
    cid = lax.axis_index("c")
    sid = lax.axis_index("s")
    wid = sid * NC + cid
    base = wid * SEG

    pltpu.sync_copy(idx_hbm.at[pl.ds(base, SEG)], idx_v)

    def to_rows(v, carry):
        rid = lax.shift_right_logical(idx_v[pl.ds(v * 16, 16)], 1)
        in_a = rid < HROWS
        a_ids_v[pl.ds(v * 16, 16)] = jnp.where(in_a, rid, -1)
        b_ids_v[pl.ds(v * 16, 16)] = jnp.where(in_a, -1, rid - HROWS)
        return carry

    lax.fori_loop(0, SEG // 16, to_rows, 0)

    srca = ta_hbm.at[plsc.Indices(a_ids_v, ignored_value=-1)]
    srcb = tb_hbm.at[plsc.Indices(b_ids_v, ignored_value=-1)]
    ca = pltpu.make_async_copy(srca, rows_v, gsem)
    cb = pltpu.make_async_copy(srcb, rows_v, gsem)
    ca.start()
    cb.start()
    ca.wait()
    cb.wait()

    pltpu.sync_copy(rows_v, out_hbm.at[pl.ds(base, SEG), :])


def kernel(token_indices, lookup):
    idx = token_indices.astype(jnp.int32)
    ta = jnp.transpose(lookup[:, : D_V // 2]).reshape(HROWS, 2 * D_M)
    tb = jnp.transpose(lookup[:, D_V // 2 :]).reshape(HROWS, 2 * D_M)
    mesh = plsc.VectorSubcoreMesh(core_axis_name="c", subcore_axis_name="s")
    k = functools.partial(
        pl.kernel,
        mesh=mesh,
        out_type=jax.ShapeDtypeStruct((B, 2 * D_M), jnp.float32),
        scratch_types=[
            pltpu.VMEM((SEG,), jnp.int32),
            pltpu.VMEM((SEG,), jnp.int32),
            pltpu.VMEM((SEG,), jnp.int32),
            pltpu.VMEM((SEG, 2 * D_M), jnp.float32),
            pltpu.SemaphoreType.DMA,
            pltpu.SemaphoreType.DMA,
        ],
    )(_body)
    pairs = k(idx, ta, tb)
    odd = (idx & 1)[:, None] == 1
    halves = jnp.where(odd, pairs[:, D_M:], pairs[:, :D_M])
    return halves.T


# speedup vs baseline: 1.1659x; 1.1659x over previous
import functools

import jax
import jax.numpy as jnp
from jax import lax
from jax.experimental import pallas as pl
from jax.experimental.pallas import tpu as pltpu
from jax.experimental.pallas import tpu_sc as plsc

D_V = 1_000_000
D_M = 64
B = 16384

NC = 2
NS = 16
NW = NC * NS
SEG = B // NW


def _body(idx_hbm, tableT_hbm, out_hbm, idx_v, row_ids_v, rows_v, sem, gsem):
    cid = lax.axis_index("c")
    sid = lax.axis_index("s")
    wid = sid * NC + cid
    base = wid * SEG

    pltpu.sync_copy(idx_hbm.at[pl.ds(base, SEG)], idx_v)

    def to_rows(v, carry):
        row_ids_v[pl.ds(v * 16, 16)] = lax.shift_right_logical(
            idx_v[pl.ds(v * 16, 16)], 1
        )
        return carry

    lax.fori_loop(0, SEG // 16, to_rows, 0)

    src = tableT_hbm.at[row_ids_v]
    pltpu.make_async_copy(src, rows_v, gsem).start()
    pltpu.make_async_copy(src, rows_v, gsem).wait()

    pltpu.sync_copy(rows_v, out_hbm.at[pl.ds(base, SEG), :])


def kernel(token_indices, lookup):
    idx = token_indices.astype(jnp.int32)
    tableT = jnp.transpose(
        lookup.reshape(D_M, D_V // 2, 2), (1, 2, 0)
    ).reshape(D_V // 2, 2 * D_M)
    mesh = plsc.VectorSubcoreMesh(core_axis_name="c", subcore_axis_name="s")
    k = functools.partial(
        pl.kernel,
        mesh=mesh,
        out_type=jax.ShapeDtypeStruct((B, 2 * D_M), jnp.float32),
        scratch_types=[
            pltpu.VMEM((SEG,), jnp.int32),
            pltpu.VMEM((SEG,), jnp.int32),
            pltpu.VMEM((SEG, 2 * D_M), jnp.float32),
            pltpu.SemaphoreType.DMA,
            pltpu.SemaphoreType.DMA,
        ],
    )(_body)
    pairs = k(idx, tableT)
    odd = (idx & 1)[:, None] == 1
    halves = jnp.where(odd, pairs[:, D_M:], pairs[:, :D_M])
    return halves.T


# speedup vs baseline: 1.6210x; 1.3903x over previous
import functools

import jax
import jax.numpy as jnp
from jax import lax
from jax.experimental import pallas as pl
from jax.experimental.pallas import tpu as pltpu
from jax.experimental.pallas import tpu_sc as plsc

D_V = 1_000_000
D_M = 64
B = 16384

NC = 2
NS = 16
NW = NC * NS
SEG = B // NW


def _body(idx_hbm, tableT_hbm, out_hbm, idx_v, rows_v, sem, gsem):
    cid = lax.axis_index("c")
    sid = lax.axis_index("s")
    wid = sid * NC + cid
    base = wid * SEG

    pltpu.sync_copy(idx_hbm.at[pl.ds(base, SEG)], idx_v)

    src = tableT_hbm.at[idx_v]
    pltpu.make_async_copy(src, rows_v, gsem).start()
    pltpu.make_async_copy(src, rows_v, gsem).wait()

    pltpu.sync_copy(rows_v, out_hbm.at[pl.ds(base, SEG), :])


def kernel(token_indices, lookup):
    idx = token_indices.astype(jnp.int32)
    tableT = jnp.pad(jnp.transpose(lookup), ((0, 0), (0, D_M)))
    mesh = plsc.VectorSubcoreMesh(core_axis_name="c", subcore_axis_name="s")
    k = functools.partial(
        pl.kernel,
        mesh=mesh,
        out_type=jax.ShapeDtypeStruct((B, 2 * D_M), jnp.float32),
        scratch_types=[
            pltpu.VMEM((SEG,), jnp.int32),
            pltpu.VMEM((SEG, 2 * D_M), jnp.float32),
            pltpu.SemaphoreType.DMA,
            pltpu.SemaphoreType.DMA,
        ],
    )(_body)
    rows = k(idx, tableT)
    return rows[:, :D_M].T
